# Initial kernel scaffold; baseline (speedup 1.0000x reference)
import jax
import jax.numpy as jnp
from jax.experimental import pallas as pl


def kernel(x, targets, protos):
    raise NotImplementedError("write your pallas kernel here")


# speedup vs baseline: 3.3385x; 3.3385x over previous
import functools

import jax
import jax.numpy as jnp
from jax.experimental import pallas as pl
from jax.experimental.pallas import tpu as pltpu

EPS = 1e-5
LAM = 0.1

W = 2048
NPIX = 8 * 128 * 128
NSTEP = NPIX // W


def _body(xref, tref, pref, oref, acc):
    g = pl.program_id(0)
    X = xref[0]
    P = pref[...]
    t = tref[0]

    ssq = jnp.sum(X * X, axis=0, keepdims=True)
    S = jax.lax.dot_general(P, X, (((1,), (0,)), ((), ())),
                            preferred_element_type=jnp.float32)
    pn2 = jnp.sum(P * P, axis=1, keepdims=True)

    iot = jax.lax.broadcasted_iota(jnp.int32, (100, 1), 0)
    O = t == iot
    dsel = jnp.sum(jnp.where(O, S, 0.0), axis=0, keepdims=True)
    pn2sel = jnp.sum(jnp.where(O, jnp.broadcast_to(pn2, O.shape), 0.0),
                     axis=0, keepdims=True)

    r = jnp.maximum(jnp.sqrt(ssq), 1e-15)
    th = jnp.tanh(r)
    scale = th / r
    nx = th * th
    denom = jnp.maximum(1.0 - nx, EPS)
    sq = pn2sel + nx - 2.0 * (scale * dsel)
    val = jnp.log(jnp.maximum(sq / denom, EPS)) - LAM * jnp.log(denom)
    m = ((t != 255) & (t != -1)).astype(jnp.float32)

    sv = jnp.sum(val * m)
    sm = jnp.sum(m)

    @pl.when(g == 0)
    def _init():
        acc[0] = 0.0
        acc[1] = 0.0

    acc[0] += sv
    acc[1] += sm

    @pl.when(g == NSTEP - 1)
    def _fin():
        oref[0, 0] = acc[0] / acc[1]


@functools.partial(jax.jit, static_argnums=())
def kernel(x, targets, protos):
    xr = x.reshape(8, 256, 128 * 128)
    tr = targets.reshape(NSTEP, 1, W)
    out = pl.pallas_call(
        _body,
        grid=(NSTEP,),
        in_specs=[
            pl.BlockSpec((1, 256, W), lambda g: (g // 8, 0, g % 8)),
            pl.BlockSpec((1, 1, W), lambda g: (g, 0, 0)),
            pl.BlockSpec((100, 256), lambda g: (0, 0)),
        ],
        out_specs=pl.BlockSpec(memory_space=pltpu.SMEM),
        out_shape=jax.ShapeDtypeStruct((1, 1), jnp.float32),
        scratch_shapes=[pltpu.SMEM((2,), jnp.float32)],
    )(xr, tr, protos)
    return out[0, 0]
